# Initial kernel scaffold; baseline (speedup 1.0000x reference)
import jax
import jax.numpy as jnp
from jax.experimental import pallas as pl


def kernel(x, edge_index, pos_edge_index, neg_edge_index, W_self_0, W_neigh_0, b_0, W_self_1, W_neigh_1, b_1, W_self_2, W_neigh_2, b_2, W_mlp1, b_mlp1, W_mlp2, b_mlp2, W_mlp3, b_mlp3):
    raise NotImplementedError("write your pallas kernel here")


# speedup vs baseline: 3.8792x; 3.8792x over previous
import functools

import jax
import jax.numpy as jnp
from jax import lax
from jax.experimental import pallas as pl
from jax.experimental.pallas import tpu as pltpu
from jax.experimental.pallas import tpu_sc as plsc

N = 10000
E = 320000
D = 128

NC = 2
NS = 16
NW = NC * NS
EDGES_PER_W = E // NW
CHUNK = 80
NCHUNK = EDGES_PER_W // CHUNK
N_PAD = 10240
ROWS_PER_TILE = N_PAD // NS
DROW = N_PAD // D

_mesh = plsc.VectorSubcoreMesh(core_axis_name="c", subcore_axis_name="s")
_sc_params = pltpu.CompilerParams(use_tc_tiling_on_sc=False,
                                  needs_layout_passes=False)


def _zero_vmem(ref, nrows, width):
  def body(r, carry):
    for j in range(width // 16):
      ref[r, pl.ds(j * 16, 16)] = jnp.zeros((16,), jnp.float32)
    return carry
  lax.fori_loop(0, nrows, body, 0)


def _sc_agg_body(with_deg, src_hbm, dst_hbm, h_hbm, *rest):
  if with_deg:
    (agg_out, deg_out, agg_sh, zb_a, srcb, dstb, rows, degacc, sem) = rest
  else:
    (agg_out, agg_sh, zb_a, srcb, dstb, rows, sem) = rest
  c = lax.axis_index("c")
  s = lax.axis_index("s")
  wid = c * NS + s

  _zero_vmem(zb_a, 128, D)
  if with_deg:
    _zero_vmem(degacc, DROW, D)
  for k in range(ROWS_PER_TILE // 128):
    pltpu.sync_copy(zb_a, agg_sh.at[pl.ds(s * ROWS_PER_TILE + k * 128, 128)])
  plsc.subcore_barrier()

  ones16 = jnp.ones((16,), jnp.float32)

  def chunk_body(j, carry):
    base = wid * EDGES_PER_W + j * CHUNK
    pltpu.sync_copy(src_hbm.at[pl.ds(base, CHUNK)], srcb)
    pltpu.sync_copy(dst_hbm.at[pl.ds(base, CHUNK)], dstb)
    pltpu.async_copy(h_hbm.at[srcb], rows, sem).wait()
    pltpu.sync_copy(rows, agg_sh.at[dstb], add=True)
    if with_deg:
      for g in range(CHUNK // 16):
        vidx = dstb[pl.ds(g * 16, 16)]
        hi = jax.lax.shift_right_logical(vidx, 7)
        lo = jax.lax.bitwise_and(vidx, 127)
        plsc.addupdate_scatter(degacc, [hi, lo], ones16)
    return carry
  lax.fori_loop(0, NCHUNK, chunk_body, 0)

  plsc.subcore_barrier()
  sl = pl.ds(s * ROWS_PER_TILE, ROWS_PER_TILE)
  pltpu.sync_copy(agg_sh.at[sl], agg_out.at[c, sl])
  if with_deg:
    pltpu.sync_copy(degacc, deg_out.at[c, s])


def _make_sc_agg(with_deg):
  out_type = [jax.ShapeDtypeStruct((NC, N_PAD, D), jnp.float32)]
  scratch = [
      pltpu.VMEM_SHARED((N_PAD, D), jnp.float32),
      pltpu.VMEM((128, D), jnp.float32),
      pltpu.VMEM((CHUNK,), jnp.int32),
      pltpu.VMEM((CHUNK,), jnp.int32),
      pltpu.VMEM((CHUNK, D), jnp.float32),
  ]
  if with_deg:
    out_type = out_type + [jax.ShapeDtypeStruct((NC, NS, DROW, D), jnp.float32)]
    scratch = scratch + [pltpu.VMEM((DROW, D), jnp.float32)]
  scratch = scratch + [pltpu.SemaphoreType.DMA]
  return pl.kernel(
      functools.partial(_sc_agg_body, with_deg),
      out_type=out_type,
      mesh=_mesh,
      scratch_types=scratch,
      compiler_params=_sc_params,
  )


_sc_agg_deg = _make_sc_agg(True)
_sc_agg = _make_sc_agg(False)


@functools.partial(
    pl.kernel,
    out_type=jax.ShapeDtypeStruct((E, D), jnp.float32),
    mesh=_mesh,
    scratch_types=[
        pltpu.VMEM((CHUNK,), jnp.int32),
        pltpu.VMEM((CHUNK,), jnp.int32),
        pltpu.VMEM((CHUNK, D), jnp.float32),
        pltpu.SemaphoreType.DMA,
    ],
    compiler_params=_sc_params,
)
def _sc_pair_gather(u_hbm, v_hbm, a_hbm, b_hbm, out_hbm, ub, vb, rows, sem):
  c = lax.axis_index("c")
  s = lax.axis_index("s")
  wid = c * NS + s

  def chunk_body(j, carry):
    base = wid * EDGES_PER_W + j * CHUNK
    pltpu.sync_copy(u_hbm.at[pl.ds(base, CHUNK)], ub)
    pltpu.sync_copy(v_hbm.at[pl.ds(base, CHUNK)], vb)
    pltpu.async_copy(a_hbm.at[ub], rows, sem).wait()
    pltpu.async_copy(b_hbm.at[vb], rows, sem, add=True).wait()
    pltpu.sync_copy(rows, out_hbm.at[pl.ds(base, CHUNK)])
    return carry
  lax.fori_loop(0, NCHUNK, chunk_body, 0)


BLK_N = 2000
BLK_E = 2000


def _deg_finalize_body(degp, out):
  d = jnp.sum(degp[...], axis=(0, 1))
  out[...] = 1.0 / jnp.maximum(d, 1.0)


def _deg_finalize(degp):
  return pl.pallas_call(
      _deg_finalize_body,
      out_shape=jax.ShapeDtypeStruct((DROW, D), jnp.float32),
  )(degp)


def _sage_tc_body(relu, aggp, recip, h, wself, wneigh, b, out):
  mean = (aggp[0] + aggp[1]) * recip[...]
  r = (jnp.dot(h[...], wself[...], preferred_element_type=jnp.float32)
       + jnp.dot(mean, wneigh[...], preferred_element_type=jnp.float32)
       + b[...])
  out[...] = jnp.maximum(r, 0.0) if relu else r


def _sage_tc(aggp, recip, h, wself, wneigh, b, relu):
  grid = (N // BLK_N,)
  return pl.pallas_call(
      functools.partial(_sage_tc_body, relu),
      grid=grid,
      in_specs=[
          pl.BlockSpec((NC, BLK_N, D), lambda m: (0, m, 0)),
          pl.BlockSpec((BLK_N, 1), lambda m: (m, 0)),
          pl.BlockSpec((BLK_N, D), lambda m: (m, 0)),
          pl.BlockSpec((D, D), lambda m: (0, 0)),
          pl.BlockSpec((D, D), lambda m: (0, 0)),
          pl.BlockSpec((1, D), lambda m: (0, 0)),
      ],
      out_specs=pl.BlockSpec((BLK_N, D), lambda m: (m, 0)),
      out_shape=jax.ShapeDtypeStruct((N, D), jnp.float32),
  )(aggp, recip, h, wself, wneigh, b)


def _sage_final_body(aggp, recip, h, wself, wneigh, b, w1t, w1b, a_out, b_out):
  mean = (aggp[0] + aggp[1]) * recip[...]
  h3 = (jnp.dot(h[...], wself[...], preferred_element_type=jnp.float32)
        + jnp.dot(mean, wneigh[...], preferred_element_type=jnp.float32)
        + b[...])
  a_out[...] = jnp.dot(h3, w1t[...], preferred_element_type=jnp.float32)
  b_out[...] = jnp.dot(h3, w1b[...], preferred_element_type=jnp.float32)


def _sage_final_tc(aggp, recip, h, wself, wneigh, b, w1t, w1b):
  grid = (N // BLK_N,)
  return pl.pallas_call(
      _sage_final_body,
      grid=grid,
      in_specs=[
          pl.BlockSpec((NC, BLK_N, D), lambda m: (0, m, 0)),
          pl.BlockSpec((BLK_N, 1), lambda m: (m, 0)),
          pl.BlockSpec((BLK_N, D), lambda m: (m, 0)),
          pl.BlockSpec((D, D), lambda m: (0, 0)),
          pl.BlockSpec((D, D), lambda m: (0, 0)),
          pl.BlockSpec((1, D), lambda m: (0, 0)),
          pl.BlockSpec((D, D), lambda m: (0, 0)),
          pl.BlockSpec((D, D), lambda m: (0, 0)),
      ],
      out_specs=[
          pl.BlockSpec((BLK_N, D), lambda m: (m, 0)),
          pl.BlockSpec((BLK_N, D), lambda m: (m, 0)),
      ],
      out_shape=[
          jax.ShapeDtypeStruct((N, D), jnp.float32),
          jax.ShapeDtypeStruct((N, D), jnp.float32),
      ],
  )(aggp, recip, h, wself, wneigh, b, w1t, w1b)


def _mlp_body(s, b1, w2, b2, w3, b3, out):
  z1 = jnp.maximum(s[...] + b1[...], 0.0)
  z2 = jnp.maximum(
      jnp.dot(z1, w2[...], preferred_element_type=jnp.float32) + b2[...], 0.0)
  out[...] = jnp.dot(z2, w3[...], preferred_element_type=jnp.float32) + b3[...]


def _mlp_tc(s, b1, w2, b2, w3, b3):
  grid = (E // BLK_E,)
  return pl.pallas_call(
      _mlp_body,
      grid=grid,
      in_specs=[
          pl.BlockSpec((BLK_E, D), lambda m: (m, 0)),
          pl.BlockSpec((1, D), lambda m: (0, 0)),
          pl.BlockSpec((D, D), lambda m: (0, 0)),
          pl.BlockSpec((1, D), lambda m: (0, 0)),
          pl.BlockSpec((D, 2), lambda m: (0, 0)),
          pl.BlockSpec((1, 2), lambda m: (0, 0)),
      ],
      out_specs=pl.BlockSpec((BLK_E, 2), lambda m: (m, 0)),
      out_shape=jax.ShapeDtypeStruct((E, 2), jnp.float32),
  )(s, b1, w2, b2, w3, b3)


def kernel(x, edge_index, pos_edge_index, neg_edge_index,
           W_self_0, W_neigh_0, b_0, W_self_1, W_neigh_1, b_1,
           W_self_2, W_neigh_2, b_2,
           W_mlp1, b_mlp1, W_mlp2, b_mlp2, W_mlp3, b_mlp3):
  b0 = b_0.reshape(1, D)
  b1l = b_1.reshape(1, D)
  b2l = b_2.reshape(1, D)
  bm1 = b_mlp1.reshape(1, D)
  bm2 = b_mlp2.reshape(1, D)
  bm3 = b_mlp3.reshape(1, 2)
  w1t = W_mlp1[:D]
  w1b = W_mlp1[D:]

  src = edge_index[0]
  dst = edge_index[1]
  pu, pv = pos_edge_index[0], pos_edge_index[1]
  nu, nv = neg_edge_index[0], neg_edge_index[1]

  aggp, degp = _sc_agg_deg(src, dst, x)
  recip = _deg_finalize(degp).reshape(N_PAD, 1)
  h = _sage_tc(aggp, recip, x, W_self_0, W_neigh_0, b0, True)
  aggp = _sc_agg(src, dst, h)[0]
  h = _sage_tc(aggp, recip, h, W_self_1, W_neigh_1, b1l, True)
  aggp = _sc_agg(src, dst, h)[0]
  A, B = _sage_final_tc(aggp, recip, h, W_self_2, W_neigh_2, b2l, w1t, w1b)

  pos_s = _sc_pair_gather(pu, pv, A, B)
  pos = _mlp_tc(pos_s, bm1, W_mlp2, bm2, W_mlp3, bm3)
  neg_s = _sc_pair_gather(nu, nv, A, B)
  neg = _mlp_tc(neg_s, bm1, W_mlp2, bm2, W_mlp3, bm3)
  return (pos, neg)


# speedup vs baseline: 4.2382x; 1.0926x over previous
import functools

import jax
import jax.numpy as jnp
from jax import lax
from jax.experimental import pallas as pl
from jax.experimental.pallas import tpu as pltpu
from jax.experimental.pallas import tpu_sc as plsc

N = 10000
E = 320000
D = 128

NC = 2
NS = 16
NW = NC * NS
EDGES_PER_W = E // NW
CHUNK = 80
NCHUNK = EDGES_PER_W // CHUNK
N_PAD = 10240
ROWS_PER_TILE = N_PAD // NS
DROW = N_PAD // D

_mesh = plsc.VectorSubcoreMesh(core_axis_name="c", subcore_axis_name="s")
_sc_params = pltpu.CompilerParams(use_tc_tiling_on_sc=False,
                                  needs_layout_passes=False)


def _zero_vmem(ref, nrows, width):
  def body(r, carry):
    for j in range(width // 16):
      ref[r, pl.ds(j * 16, 16)] = jnp.zeros((16,), jnp.float32)
    return carry
  lax.fori_loop(0, nrows, body, 0)


def _sc_agg_body(with_deg, src_hbm, dst_hbm, h_hbm, *rest):
  if with_deg:
    (agg_out, deg_out, agg_sh, zb_a, srcb, dstb, rows, degacc,
     sg0, sg1, ss0, ss1) = rest
  else:
    (agg_out, agg_sh, zb_a, srcb, dstb, rows, sg0, sg1, ss0, ss1) = rest
  semg = (sg0, sg1)
  sems = (ss0, ss1)
  c = lax.axis_index("c")
  s = lax.axis_index("s")
  wid = c * NS + s
  ebase = wid * EDGES_PER_W

  _zero_vmem(zb_a, 128, D)
  if with_deg:
    _zero_vmem(degacc, DROW, D)
  for k in range(ROWS_PER_TILE // 128):
    pltpu.sync_copy(zb_a, agg_sh.at[pl.ds(s * ROWS_PER_TILE + k * 128, 128)])
  plsc.subcore_barrier()

  ones16 = jnp.ones((16,), jnp.float32)

  def load_idx(j, b):
    pltpu.sync_copy(src_hbm.at[pl.ds(ebase + j * CHUNK, CHUNK)], srcb.at[b])
    pltpu.sync_copy(dst_hbm.at[pl.ds(ebase + j * CHUNK, CHUNK)], dstb.at[b])

  def issue_gather(b):
    return pltpu.async_copy(h_hbm.at[srcb.at[b]], rows.at[b], semg[b])

  def deg_update(b):
    for g in range(CHUNK // 16):
      vidx = dstb[b, pl.ds(g * 16, 16)]
      hi = jax.lax.shift_right_logical(vidx, 7)
      lo = jax.lax.bitwise_and(vidx, 127)
      plsc.addupdate_scatter(degacc, [hi, lo], ones16)

  def step(j, b, first, prefetch):
    b2 = 1 - b
    if not first:
      pltpu.make_async_copy(rows.at[b2], agg_sh.at[dstb.at[b2]],
                            sems[b2]).wait()
    if prefetch:
      load_idx(j + 1, b2)
      issue_gather(b2)
    pltpu.make_async_copy(h_hbm.at[srcb.at[b]], rows.at[b], semg[b]).wait()
    pltpu.async_copy(rows.at[b], agg_sh.at[dstb.at[b]], sems[b], add=True)
    if with_deg:
      deg_update(b)

  load_idx(0, 0)
  issue_gather(0)
  step(0, 0, True, True)

  def two_steps(j2, carry):
    j = 1 + 2 * j2
    step(j, 1, False, True)
    step(j + 1, 0, False, True)
    return carry
  npairs = (NCHUNK - 3) // 2
  lax.fori_loop(0, npairs, two_steps, 0)

  for j in range(2 * npairs + 1, NCHUNK):
    step(j, j % 2, False, j < NCHUNK - 1)
  bl = (NCHUNK - 1) % 2
  pltpu.make_async_copy(rows.at[bl], agg_sh.at[dstb.at[bl]], sems[bl]).wait()

  plsc.subcore_barrier()
  sl = pl.ds(s * ROWS_PER_TILE, ROWS_PER_TILE)
  pltpu.sync_copy(agg_sh.at[sl], agg_out.at[c, sl])
  if with_deg:
    pltpu.sync_copy(degacc, deg_out.at[c, s])


def _make_sc_agg(with_deg):
  out_type = [jax.ShapeDtypeStruct((NC, N_PAD, D), jnp.float32)]
  scratch = [
      pltpu.VMEM_SHARED((N_PAD, D), jnp.float32),
      pltpu.VMEM((128, D), jnp.float32),
      pltpu.VMEM((2, CHUNK), jnp.int32),
      pltpu.VMEM((2, CHUNK), jnp.int32),
      pltpu.VMEM((2, CHUNK, D), jnp.float32),
  ]
  if with_deg:
    out_type = out_type + [jax.ShapeDtypeStruct((NC, NS, DROW, D), jnp.float32)]
    scratch = scratch + [pltpu.VMEM((DROW, D), jnp.float32)]
  scratch = scratch + [pltpu.SemaphoreType.DMA] * 4
  return pl.kernel(
      functools.partial(_sc_agg_body, with_deg),
      out_type=out_type,
      mesh=_mesh,
      scratch_types=scratch,
      compiler_params=_sc_params,
  )


_sc_agg_deg = _make_sc_agg(True)
_sc_agg = _make_sc_agg(False)


E2 = 2 * E
EDGES_PER_W2 = E2 // NW
NCHUNK2 = EDGES_PER_W2 // CHUNK


@functools.partial(
    pl.kernel,
    out_type=jax.ShapeDtypeStruct((E2, D), jnp.float32),
    mesh=_mesh,
    scratch_types=[
        pltpu.VMEM((2, CHUNK), jnp.int32),
        pltpu.VMEM((2, CHUNK), jnp.int32),
        pltpu.VMEM((2, CHUNK, D), jnp.float32),
        pltpu.SemaphoreType.DMA,
        pltpu.SemaphoreType.DMA,
        pltpu.SemaphoreType.DMA,
        pltpu.SemaphoreType.DMA,
    ],
    compiler_params=_sc_params,
)
def _sc_pair_gather(u_hbm, v_hbm, a_hbm, b_hbm, out_hbm, ub, vb, rows,
                    sg0, sg1, sw0, sw1):
  semg = (sg0, sg1)
  semw = (sw0, sw1)
  c = lax.axis_index("c")
  s = lax.axis_index("s")
  wid = c * NS + s
  ebase = wid * EDGES_PER_W2

  def load_idx(j, b):
    pltpu.sync_copy(u_hbm.at[pl.ds(ebase + j * CHUNK, CHUNK)], ub.at[b])
    pltpu.sync_copy(v_hbm.at[pl.ds(ebase + j * CHUNK, CHUNK)], vb.at[b])

  def step(j, b, first, prefetch):
    b2 = 1 - b
    if not first:
      pltpu.make_async_copy(
          rows.at[b2], out_hbm.at[pl.ds(ebase + (j - 1) * CHUNK, CHUNK)],
          semw[b2]).wait()
    if prefetch:
      load_idx(j + 1, b2)
      pltpu.async_copy(a_hbm.at[ub.at[b2]], rows.at[b2], semg[b2])
    pltpu.make_async_copy(a_hbm.at[ub.at[b]], rows.at[b], semg[b]).wait()
    pltpu.async_copy(b_hbm.at[vb.at[b]], rows.at[b], semg[b], add=True).wait()
    pltpu.async_copy(rows.at[b], out_hbm.at[pl.ds(ebase + j * CHUNK, CHUNK)],
                     semw[b])

  load_idx(0, 0)
  pltpu.async_copy(a_hbm.at[ub.at[0]], rows.at[0], semg[0])
  step(0, 0, True, True)

  def two_steps(j2, carry):
    j = 1 + 2 * j2
    step(j, 1, False, True)
    step(j + 1, 0, False, True)
    return carry
  npairs = (NCHUNK2 - 3) // 2
  lax.fori_loop(0, npairs, two_steps, 0)

  for j in range(2 * npairs + 1, NCHUNK2):
    step(j, j % 2, False, j < NCHUNK2 - 1)
  bl = (NCHUNK2 - 1) % 2
  pltpu.make_async_copy(
      rows.at[bl], out_hbm.at[pl.ds(ebase + (NCHUNK2 - 1) * CHUNK, CHUNK)],
      semw[bl]).wait()


BLK_N = 2000
BLK_E = 2000


def _deg_finalize_body(degp, out):
  d = jnp.sum(degp[...], axis=(0, 1))
  out[...] = 1.0 / jnp.maximum(d, 1.0)


def _deg_finalize(degp):
  return pl.pallas_call(
      _deg_finalize_body,
      out_shape=jax.ShapeDtypeStruct((DROW, D), jnp.float32),
  )(degp)


def _sage_tc_body(relu, aggp, recip, h, wself, wneigh, b, out):
  mean = (aggp[0] + aggp[1]) * recip[...]
  r = (jnp.dot(h[...], wself[...], preferred_element_type=jnp.float32)
       + jnp.dot(mean, wneigh[...], preferred_element_type=jnp.float32)
       + b[...])
  out[...] = jnp.maximum(r, 0.0) if relu else r


def _sage_tc(aggp, recip, h, wself, wneigh, b, relu):
  grid = (N // BLK_N,)
  return pl.pallas_call(
      functools.partial(_sage_tc_body, relu),
      grid=grid,
      in_specs=[
          pl.BlockSpec((NC, BLK_N, D), lambda m: (0, m, 0)),
          pl.BlockSpec((BLK_N, 1), lambda m: (m, 0)),
          pl.BlockSpec((BLK_N, D), lambda m: (m, 0)),
          pl.BlockSpec((D, D), lambda m: (0, 0)),
          pl.BlockSpec((D, D), lambda m: (0, 0)),
          pl.BlockSpec((1, D), lambda m: (0, 0)),
      ],
      out_specs=pl.BlockSpec((BLK_N, D), lambda m: (m, 0)),
      out_shape=jax.ShapeDtypeStruct((N, D), jnp.float32),
  )(aggp, recip, h, wself, wneigh, b)


def _sage_final_body(aggp, recip, h, wself, wneigh, b, w1t, w1b, a_out, b_out):
  mean = (aggp[0] + aggp[1]) * recip[...]
  h3 = (jnp.dot(h[...], wself[...], preferred_element_type=jnp.float32)
        + jnp.dot(mean, wneigh[...], preferred_element_type=jnp.float32)
        + b[...])
  a_out[...] = jnp.dot(h3, w1t[...], preferred_element_type=jnp.float32)
  b_out[...] = jnp.dot(h3, w1b[...], preferred_element_type=jnp.float32)


def _sage_final_tc(aggp, recip, h, wself, wneigh, b, w1t, w1b):
  grid = (N // BLK_N,)
  return pl.pallas_call(
      _sage_final_body,
      grid=grid,
      in_specs=[
          pl.BlockSpec((NC, BLK_N, D), lambda m: (0, m, 0)),
          pl.BlockSpec((BLK_N, 1), lambda m: (m, 0)),
          pl.BlockSpec((BLK_N, D), lambda m: (m, 0)),
          pl.BlockSpec((D, D), lambda m: (0, 0)),
          pl.BlockSpec((D, D), lambda m: (0, 0)),
          pl.BlockSpec((1, D), lambda m: (0, 0)),
          pl.BlockSpec((D, D), lambda m: (0, 0)),
          pl.BlockSpec((D, D), lambda m: (0, 0)),
      ],
      out_specs=[
          pl.BlockSpec((BLK_N, D), lambda m: (m, 0)),
          pl.BlockSpec((BLK_N, D), lambda m: (m, 0)),
      ],
      out_shape=[
          jax.ShapeDtypeStruct((N, D), jnp.float32),
          jax.ShapeDtypeStruct((N, D), jnp.float32),
      ],
  )(aggp, recip, h, wself, wneigh, b, w1t, w1b)


def _mlp_body(s, b1, w2, b2, w3, b3, out):
  z1 = jnp.maximum(s[...] + b1[...], 0.0)
  z2 = jnp.maximum(
      jnp.dot(z1, w2[...], preferred_element_type=jnp.float32) + b2[...], 0.0)
  out[...] = jnp.dot(z2, w3[...], preferred_element_type=jnp.float32) + b3[...]


def _mlp_tc(s, b1, w2, b2, w3, b3):
  grid = (E2 // BLK_E,)
  return pl.pallas_call(
      _mlp_body,
      grid=grid,
      in_specs=[
          pl.BlockSpec((BLK_E, D), lambda m: (m, 0)),
          pl.BlockSpec((1, D), lambda m: (0, 0)),
          pl.BlockSpec((D, D), lambda m: (0, 0)),
          pl.BlockSpec((1, D), lambda m: (0, 0)),
          pl.BlockSpec((D, 2), lambda m: (0, 0)),
          pl.BlockSpec((1, 2), lambda m: (0, 0)),
      ],
      out_specs=pl.BlockSpec((BLK_E, 2), lambda m: (m, 0)),
      out_shape=jax.ShapeDtypeStruct((E2, 2), jnp.float32),
  )(s, b1, w2, b2, w3, b3)


def kernel(x, edge_index, pos_edge_index, neg_edge_index,
           W_self_0, W_neigh_0, b_0, W_self_1, W_neigh_1, b_1,
           W_self_2, W_neigh_2, b_2,
           W_mlp1, b_mlp1, W_mlp2, b_mlp2, W_mlp3, b_mlp3):
  b0 = b_0.reshape(1, D)
  b1l = b_1.reshape(1, D)
  b2l = b_2.reshape(1, D)
  bm1 = b_mlp1.reshape(1, D)
  bm2 = b_mlp2.reshape(1, D)
  bm3 = b_mlp3.reshape(1, 2)
  w1t = W_mlp1[:D]
  w1b = W_mlp1[D:]

  src = edge_index[0]
  dst = edge_index[1]
  u_all = jnp.concatenate([pos_edge_index[0], neg_edge_index[0]])
  v_all = jnp.concatenate([pos_edge_index[1], neg_edge_index[1]])

  aggp, degp = _sc_agg_deg(src, dst, x)
  recip = _deg_finalize(degp).reshape(N_PAD, 1)
  h = _sage_tc(aggp, recip, x, W_self_0, W_neigh_0, b0, True)
  aggp = _sc_agg(src, dst, h)[0]
  h = _sage_tc(aggp, recip, h, W_self_1, W_neigh_1, b1l, True)
  aggp = _sc_agg(src, dst, h)[0]
  A, B = _sage_final_tc(aggp, recip, h, W_self_2, W_neigh_2, b2l, w1t, w1b)

  s_all = _sc_pair_gather(u_all, v_all, A, B)
  out_all = _mlp_tc(s_all, bm1, W_mlp2, bm2, W_mlp3, bm3)
  return (out_all[:E], out_all[E:])


# speedup vs baseline: 5.2848x; 1.2469x over previous
import functools

import jax
import jax.numpy as jnp
from jax import lax
from jax.experimental import pallas as pl
from jax.experimental.pallas import tpu as pltpu
from jax.experimental.pallas import tpu_sc as plsc

N = 10000
E = 320000
D = 128

NC = 2
NS = 16
NW = NC * NS
EDGES_PER_W = E // NW
CHUNK = 80
NCHUNK = EDGES_PER_W // CHUNK
N_PAD = 10240
ROWS_PER_TILE = N_PAD // NS
DROW = N_PAD // D

_mesh = plsc.VectorSubcoreMesh(core_axis_name="c", subcore_axis_name="s")
_sc_params = pltpu.CompilerParams(use_tc_tiling_on_sc=False,
                                  needs_layout_passes=False)


def _zero_vmem(ref, nrows, width):
  def body(r, carry):
    for j in range(width // 16):
      ref[r, pl.ds(j * 16, 16)] = jnp.zeros((16,), jnp.float32)
    return carry
  lax.fori_loop(0, nrows, body, 0)


def _sc_agg_body(with_deg, src_hbm, dst_hbm, h_hbm, *rest):
  if with_deg:
    (agg_out, deg_out, agg_sh, zb_a, srcb, dstb, rows, degacc,
     sg0, sg1, ss0, ss1) = rest
  else:
    (agg_out, agg_sh, zb_a, srcb, dstb, rows, sg0, sg1, ss0, ss1) = rest
  semg = (sg0, sg1)
  sems = (ss0, ss1)
  c = lax.axis_index("c")
  s = lax.axis_index("s")
  wid = c * NS + s
  ebase = wid * EDGES_PER_W

  _zero_vmem(zb_a, 128, D)
  if with_deg:
    _zero_vmem(degacc, DROW, D)
  for k in range(ROWS_PER_TILE // 128):
    pltpu.sync_copy(zb_a, agg_sh.at[pl.ds(s * ROWS_PER_TILE + k * 128, 128)])
  plsc.subcore_barrier()

  ones16 = jnp.ones((16,), jnp.float32)

  def load_idx(j, b):
    pltpu.sync_copy(src_hbm.at[pl.ds(ebase + j * CHUNK, CHUNK)], srcb.at[b])
    pltpu.sync_copy(dst_hbm.at[pl.ds(ebase + j * CHUNK, CHUNK)], dstb.at[b])

  def issue_gather(b):
    return pltpu.async_copy(h_hbm.at[srcb.at[b]], rows.at[b], semg[b])

  def deg_update(b):
    for g in range(CHUNK // 16):
      vidx = dstb[b, pl.ds(g * 16, 16)]
      hi = jax.lax.shift_right_logical(vidx, 7)
      lo = jax.lax.bitwise_and(vidx, 127)
      plsc.addupdate_scatter(degacc, [hi, lo], ones16)

  def step(j, b, first, prefetch):
    b2 = 1 - b
    if not first:
      pltpu.make_async_copy(rows.at[b2], agg_sh.at[dstb.at[b2]],
                            sems[b2]).wait()
    if prefetch:
      load_idx(j + 1, b2)
      issue_gather(b2)
    pltpu.make_async_copy(h_hbm.at[srcb.at[b]], rows.at[b], semg[b]).wait()
    pltpu.async_copy(rows.at[b], agg_sh.at[dstb.at[b]], sems[b], add=True)
    if with_deg:
      deg_update(b)

  load_idx(0, 0)
  issue_gather(0)
  step(0, 0, True, True)

  def two_steps(j2, carry):
    j = 1 + 2 * j2
    step(j, 1, False, True)
    step(j + 1, 0, False, True)
    return carry
  npairs = (NCHUNK - 3) // 2
  lax.fori_loop(0, npairs, two_steps, 0)

  for j in range(2 * npairs + 1, NCHUNK):
    step(j, j % 2, False, j < NCHUNK - 1)
  bl = (NCHUNK - 1) % 2
  pltpu.make_async_copy(rows.at[bl], agg_sh.at[dstb.at[bl]], sems[bl]).wait()

  plsc.subcore_barrier()
  sl = pl.ds(s * ROWS_PER_TILE, ROWS_PER_TILE)
  pltpu.sync_copy(agg_sh.at[sl], agg_out.at[c, sl])
  if with_deg:
    pltpu.sync_copy(degacc, deg_out.at[c, s])


def _make_sc_agg(with_deg):
  out_type = [jax.ShapeDtypeStruct((NC, N_PAD, D), jnp.float32)]
  scratch = [
      pltpu.VMEM_SHARED((N_PAD, D), jnp.float32),
      pltpu.VMEM((128, D), jnp.float32),
      pltpu.VMEM((2, CHUNK), jnp.int32),
      pltpu.VMEM((2, CHUNK), jnp.int32),
      pltpu.VMEM((2, CHUNK, D), jnp.float32),
  ]
  if with_deg:
    out_type = out_type + [jax.ShapeDtypeStruct((NC, NS, DROW, D), jnp.float32)]
    scratch = scratch + [pltpu.VMEM((DROW, D), jnp.float32)]
  scratch = scratch + [pltpu.SemaphoreType.DMA] * 4
  return pl.kernel(
      functools.partial(_sc_agg_body, with_deg),
      out_type=out_type,
      mesh=_mesh,
      scratch_types=scratch,
      compiler_params=_sc_params,
  )


_sc_agg_deg = _make_sc_agg(True)
_sc_agg = _make_sc_agg(False)


@functools.partial(
    pl.kernel,
    out_type=jax.ShapeDtypeStruct((E, D), jnp.float32),
    mesh=_mesh,
    scratch_types=[
        pltpu.VMEM((2, CHUNK), jnp.int32),
        pltpu.VMEM((2, CHUNK), jnp.int32),
        pltpu.VMEM((2, CHUNK, D), jnp.float32),
        pltpu.SemaphoreType.DMA,
        pltpu.SemaphoreType.DMA,
        pltpu.SemaphoreType.DMA,
        pltpu.SemaphoreType.DMA,
    ],
    compiler_params=_sc_params,
)
def _sc_pair_gather(u_hbm, v_hbm, a_hbm, b_hbm, out_hbm, ub, vb, rows,
                    sg0, sg1, sw0, sw1):
  semg = (sg0, sg1)
  semw = (sw0, sw1)
  c = lax.axis_index("c")
  s = lax.axis_index("s")
  wid = c * NS + s
  ebase = wid * EDGES_PER_W

  def load_idx(j, b):
    pltpu.sync_copy(u_hbm.at[pl.ds(ebase + j * CHUNK, CHUNK)], ub.at[b])
    pltpu.sync_copy(v_hbm.at[pl.ds(ebase + j * CHUNK, CHUNK)], vb.at[b])

  def step(j, b, first, prefetch):
    b2 = 1 - b
    if not first:
      pltpu.make_async_copy(
          rows.at[b2], out_hbm.at[pl.ds(ebase + (j - 1) * CHUNK, CHUNK)],
          semw[b2]).wait()
    if prefetch:
      load_idx(j + 1, b2)
      pltpu.async_copy(a_hbm.at[ub.at[b2]], rows.at[b2], semg[b2])
    pltpu.make_async_copy(a_hbm.at[ub.at[b]], rows.at[b], semg[b]).wait()
    pltpu.async_copy(b_hbm.at[vb.at[b]], rows.at[b], semg[b], add=True).wait()
    pltpu.async_copy(rows.at[b], out_hbm.at[pl.ds(ebase + j * CHUNK, CHUNK)],
                     semw[b])

  load_idx(0, 0)
  pltpu.async_copy(a_hbm.at[ub.at[0]], rows.at[0], semg[0])
  step(0, 0, True, True)

  def two_steps(j2, carry):
    j = 1 + 2 * j2
    step(j, 1, False, True)
    step(j + 1, 0, False, True)
    return carry
  npairs = (NCHUNK - 3) // 2
  lax.fori_loop(0, npairs, two_steps, 0)

  for j in range(2 * npairs + 1, NCHUNK):
    step(j, j % 2, False, j < NCHUNK - 1)
  bl = (NCHUNK - 1) % 2
  pltpu.make_async_copy(
      rows.at[bl], out_hbm.at[pl.ds(ebase + (NCHUNK - 1) * CHUNK, CHUNK)],
      semw[bl]).wait()


BLK_N = 2000
BLK_E = 2000


def _deg_finalize_body(degp, out):
  d = jnp.sum(degp[...], axis=(0, 1))
  out[...] = 1.0 / jnp.maximum(d, 1.0)


def _deg_finalize(degp):
  return pl.pallas_call(
      _deg_finalize_body,
      out_shape=jax.ShapeDtypeStruct((DROW, D), jnp.float32),
  )(degp)


def _sage_tc_body(relu, aggp, recip, h, wself, wneigh, b, out):
  mean = (aggp[0] + aggp[1]) * recip[...]
  r = (jnp.dot(h[...], wself[...], preferred_element_type=jnp.float32)
       + jnp.dot(mean, wneigh[...], preferred_element_type=jnp.float32)
       + b[...])
  out[...] = jnp.maximum(r, 0.0) if relu else r


def _sage_tc(aggp, recip, h, wself, wneigh, b, relu):
  grid = (N // BLK_N,)
  return pl.pallas_call(
      functools.partial(_sage_tc_body, relu),
      grid=grid,
      in_specs=[
          pl.BlockSpec((NC, BLK_N, D), lambda m: (0, m, 0)),
          pl.BlockSpec((BLK_N, 1), lambda m: (m, 0)),
          pl.BlockSpec((BLK_N, D), lambda m: (m, 0)),
          pl.BlockSpec((D, D), lambda m: (0, 0)),
          pl.BlockSpec((D, D), lambda m: (0, 0)),
          pl.BlockSpec((1, D), lambda m: (0, 0)),
      ],
      out_specs=pl.BlockSpec((BLK_N, D), lambda m: (m, 0)),
      out_shape=jax.ShapeDtypeStruct((N, D), jnp.float32),
  )(aggp, recip, h, wself, wneigh, b)


def _sage_final_body(aggp, recip, h, wself, wneigh, b, w1t, w1b, a_out, b_out):
  mean = (aggp[0] + aggp[1]) * recip[...]
  h3 = (jnp.dot(h[...], wself[...], preferred_element_type=jnp.float32)
        + jnp.dot(mean, wneigh[...], preferred_element_type=jnp.float32)
        + b[...])
  a_out[...] = jnp.dot(h3, w1t[...], preferred_element_type=jnp.float32)
  b_out[...] = jnp.dot(h3, w1b[...], preferred_element_type=jnp.float32)


def _sage_final_tc(aggp, recip, h, wself, wneigh, b, w1t, w1b):
  grid = (N // BLK_N,)
  return pl.pallas_call(
      _sage_final_body,
      grid=grid,
      in_specs=[
          pl.BlockSpec((NC, BLK_N, D), lambda m: (0, m, 0)),
          pl.BlockSpec((BLK_N, 1), lambda m: (m, 0)),
          pl.BlockSpec((BLK_N, D), lambda m: (m, 0)),
          pl.BlockSpec((D, D), lambda m: (0, 0)),
          pl.BlockSpec((D, D), lambda m: (0, 0)),
          pl.BlockSpec((1, D), lambda m: (0, 0)),
          pl.BlockSpec((D, D), lambda m: (0, 0)),
          pl.BlockSpec((D, D), lambda m: (0, 0)),
      ],
      out_specs=[
          pl.BlockSpec((BLK_N, D), lambda m: (m, 0)),
          pl.BlockSpec((BLK_N, D), lambda m: (m, 0)),
      ],
      out_shape=[
          jax.ShapeDtypeStruct((N, D), jnp.float32),
          jax.ShapeDtypeStruct((N, D), jnp.float32),
      ],
  )(aggp, recip, h, wself, wneigh, b, w1t, w1b)


def _mlp_body(s, b1, w2, b2, w3, b3, out):
  z1 = jnp.maximum(s[...] + b1[...], 0.0)
  z2 = jnp.maximum(
      jnp.dot(z1, w2[...], preferred_element_type=jnp.float32) + b2[...], 0.0)
  out[...] = jnp.dot(z2, w3[...], preferred_element_type=jnp.float32) + b3[...]


def _mlp_tc(s, b1, w2, b2, w3, b3):
  grid = (E // BLK_E,)
  return pl.pallas_call(
      _mlp_body,
      grid=grid,
      in_specs=[
          pl.BlockSpec((BLK_E, D), lambda m: (m, 0)),
          pl.BlockSpec((1, D), lambda m: (0, 0)),
          pl.BlockSpec((D, D), lambda m: (0, 0)),
          pl.BlockSpec((1, D), lambda m: (0, 0)),
          pl.BlockSpec((D, 2), lambda m: (0, 0)),
          pl.BlockSpec((1, 2), lambda m: (0, 0)),
      ],
      out_specs=pl.BlockSpec((BLK_E, 2), lambda m: (m, 0)),
      out_shape=jax.ShapeDtypeStruct((E, 2), jnp.float32),
  )(s, b1, w2, b2, w3, b3)


def kernel(x, edge_index, pos_edge_index, neg_edge_index,
           W_self_0, W_neigh_0, b_0, W_self_1, W_neigh_1, b_1,
           W_self_2, W_neigh_2, b_2,
           W_mlp1, b_mlp1, W_mlp2, b_mlp2, W_mlp3, b_mlp3):
  b0 = b_0.reshape(1, D)
  b1l = b_1.reshape(1, D)
  b2l = b_2.reshape(1, D)
  bm1 = b_mlp1.reshape(1, D)
  bm2 = b_mlp2.reshape(1, D)
  bm3 = b_mlp3.reshape(1, 2)
  w1t = W_mlp1[:D]
  w1b = W_mlp1[D:]

  src = edge_index[0]
  dst = edge_index[1]
  pu, pv = pos_edge_index[0], pos_edge_index[1]
  nu, nv = neg_edge_index[0], neg_edge_index[1]

  aggp, degp = _sc_agg_deg(src, dst, x)
  recip = _deg_finalize(degp).reshape(N_PAD, 1)
  h = _sage_tc(aggp, recip, x, W_self_0, W_neigh_0, b0, True)
  aggp = _sc_agg(src, dst, h)[0]
  h = _sage_tc(aggp, recip, h, W_self_1, W_neigh_1, b1l, True)
  aggp = _sc_agg(src, dst, h)[0]
  A, B = _sage_final_tc(aggp, recip, h, W_self_2, W_neigh_2, b2l, w1t, w1b)

  s_pos = _sc_pair_gather(pu, pv, A, B)
  s_neg = _sc_pair_gather(nu, nv, A, B)
  pos = _mlp_tc(s_pos, bm1, W_mlp2, bm2, W_mlp3, bm3)
  neg = _mlp_tc(s_neg, bm1, W_mlp2, bm2, W_mlp3, bm3)
  return (pos, neg)


# speedup vs baseline: 5.9275x; 1.1216x over previous
import functools

import jax
import jax.numpy as jnp
from jax import lax
from jax.experimental import pallas as pl
from jax.experimental.pallas import tpu as pltpu
from jax.experimental.pallas import tpu_sc as plsc

N = 10000
E = 320000
D = 128

NC = 2
NS = 16
NW = NC * NS
EDGES_PER_W = E // NW
CHUNK = 80
NCHUNK = EDGES_PER_W // CHUNK
N_PAD = 10240
ROWS_PER_TILE = N_PAD // NS
DROW = N_PAD // D

_mesh = plsc.VectorSubcoreMesh(core_axis_name="c", subcore_axis_name="s")
_sc_params = pltpu.CompilerParams(use_tc_tiling_on_sc=False,
                                  needs_layout_passes=False)


def _zero_vmem(ref, nrows, width):
  def body(r, carry):
    for j in range(width // 16):
      ref[r, pl.ds(j * 16, 16)] = jnp.zeros((16,), jnp.float32)
    return carry
  lax.fori_loop(0, nrows, body, 0)


def _sc_agg_body(with_deg, src_hbm, dst_hbm, h_hbm, *rest):
  if with_deg:
    (agg_out, deg_out, agg_sh, zb_a, srcb, dstb, rows, degacc,
     sg0, sg1, ss0, ss1) = rest
  else:
    (agg_out, agg_sh, zb_a, srcb, dstb, rows, sg0, sg1, ss0, ss1) = rest
  semg = (sg0, sg1)
  sems = (ss0, ss1)
  c = lax.axis_index("c")
  s = lax.axis_index("s")
  wid = c * NS + s
  ebase = wid * EDGES_PER_W

  _zero_vmem(zb_a, 128, D)
  if with_deg:
    _zero_vmem(degacc, DROW, D)
  for k in range(ROWS_PER_TILE // 128):
    pltpu.sync_copy(zb_a, agg_sh.at[pl.ds(s * ROWS_PER_TILE + k * 128, 128)])
  plsc.subcore_barrier()

  ones16 = jnp.ones((16,), jnp.float32)

  def load_idx(j, b):
    pltpu.sync_copy(src_hbm.at[pl.ds(ebase + j * CHUNK, CHUNK)], srcb.at[b])
    pltpu.sync_copy(dst_hbm.at[pl.ds(ebase + j * CHUNK, CHUNK)], dstb.at[b])

  def issue_gather(b):
    return pltpu.async_copy(h_hbm.at[srcb.at[b]], rows.at[b], semg[b])

  def deg_update(b):
    for g in range(CHUNK // 16):
      vidx = dstb[b, pl.ds(g * 16, 16)]
      hi = jax.lax.shift_right_logical(vidx, 7)
      lo = jax.lax.bitwise_and(vidx, 127)
      plsc.addupdate_scatter(degacc, [hi, lo], ones16)

  def step(j, b, first, prefetch):
    b2 = 1 - b
    if not first:
      pltpu.make_async_copy(rows.at[b2], agg_sh.at[dstb.at[b2]],
                            sems[b2]).wait()
    if prefetch:
      load_idx(j + 1, b2)
      issue_gather(b2)
    pltpu.make_async_copy(h_hbm.at[srcb.at[b]], rows.at[b], semg[b]).wait()
    pltpu.async_copy(rows.at[b], agg_sh.at[dstb.at[b]], sems[b], add=True)
    if with_deg:
      deg_update(b)

  load_idx(0, 0)
  issue_gather(0)
  step(0, 0, True, True)

  def two_steps(j2, carry):
    j = 1 + 2 * j2
    step(j, 1, False, True)
    step(j + 1, 0, False, True)
    return carry
  npairs = (NCHUNK - 3) // 2
  lax.fori_loop(0, npairs, two_steps, 0)

  for j in range(2 * npairs + 1, NCHUNK):
    step(j, j % 2, False, j < NCHUNK - 1)
  bl = (NCHUNK - 1) % 2
  pltpu.make_async_copy(rows.at[bl], agg_sh.at[dstb.at[bl]], sems[bl]).wait()

  plsc.subcore_barrier()
  sl = pl.ds(s * ROWS_PER_TILE, ROWS_PER_TILE)
  pltpu.sync_copy(agg_sh.at[sl], agg_out.at[c, sl])
  if with_deg:
    pltpu.sync_copy(degacc, deg_out.at[c, s])


def _make_sc_agg(with_deg):
  out_type = [jax.ShapeDtypeStruct((NC, N_PAD, D), jnp.float32)]
  scratch = [
      pltpu.VMEM_SHARED((N_PAD, D), jnp.float32),
      pltpu.VMEM((128, D), jnp.float32),
      pltpu.VMEM((2, CHUNK), jnp.int32),
      pltpu.VMEM((2, CHUNK), jnp.int32),
      pltpu.VMEM((2, CHUNK, D), jnp.float32),
  ]
  if with_deg:
    out_type = out_type + [jax.ShapeDtypeStruct((NC, NS, DROW, D), jnp.float32)]
    scratch = scratch + [pltpu.VMEM((DROW, D), jnp.float32)]
  scratch = scratch + [pltpu.SemaphoreType.DMA] * 4
  return pl.kernel(
      functools.partial(_sc_agg_body, with_deg),
      out_type=out_type,
      mesh=_mesh,
      scratch_types=scratch,
      compiler_params=_sc_params,
  )


_sc_agg_deg = _make_sc_agg(True)
_sc_agg = _make_sc_agg(False)


@functools.partial(
    pl.kernel,
    out_type=jax.ShapeDtypeStruct((E, D), jnp.float32),
    mesh=_mesh,
    scratch_types=[
        pltpu.VMEM((2, CHUNK), jnp.int32),
        pltpu.VMEM((2, CHUNK), jnp.int32),
        pltpu.VMEM((2, CHUNK, D), jnp.float32),
        pltpu.SemaphoreType.DMA,
        pltpu.SemaphoreType.DMA,
        pltpu.SemaphoreType.DMA,
        pltpu.SemaphoreType.DMA,
    ],
    compiler_params=_sc_params,
)
def _sc_pair_gather(u_hbm, v_hbm, a_hbm, b_hbm, out_hbm, ub, vb, rows,
                    sg0, sg1, sw0, sw1):
  semg = (sg0, sg1)
  semw = (sw0, sw1)
  c = lax.axis_index("c")
  s = lax.axis_index("s")
  wid = c * NS + s
  ebase = wid * EDGES_PER_W

  def load_idx(j, b):
    pltpu.sync_copy(u_hbm.at[pl.ds(ebase + j * CHUNK, CHUNK)], ub.at[b])
    pltpu.sync_copy(v_hbm.at[pl.ds(ebase + j * CHUNK, CHUNK)], vb.at[b])

  def step(j, b, first, prefetch):
    b2 = 1 - b
    if not first:
      pltpu.make_async_copy(
          rows.at[b2], out_hbm.at[pl.ds(ebase + (j - 1) * CHUNK, CHUNK)],
          semw[b2]).wait()
    if prefetch:
      load_idx(j + 1, b2)
      pltpu.async_copy(a_hbm.at[ub.at[b2]], rows.at[b2], semg[b2])
    pltpu.make_async_copy(a_hbm.at[ub.at[b]], rows.at[b], semg[b]).wait()
    pltpu.async_copy(b_hbm.at[vb.at[b]], rows.at[b], semg[b], add=True).wait()
    pltpu.async_copy(rows.at[b], out_hbm.at[pl.ds(ebase + j * CHUNK, CHUNK)],
                     semw[b])

  load_idx(0, 0)
  pltpu.async_copy(a_hbm.at[ub.at[0]], rows.at[0], semg[0])
  step(0, 0, True, True)

  def two_steps(j2, carry):
    j = 1 + 2 * j2
    step(j, 1, False, True)
    step(j + 1, 0, False, True)
    return carry
  npairs = (NCHUNK - 3) // 2
  lax.fori_loop(0, npairs, two_steps, 0)

  for j in range(2 * npairs + 1, NCHUNK):
    step(j, j % 2, False, j < NCHUNK - 1)
  bl = (NCHUNK - 1) % 2
  pltpu.make_async_copy(
      rows.at[bl], out_hbm.at[pl.ds(ebase + (NCHUNK - 1) * CHUNK, CHUNK)],
      semw[bl]).wait()


BLK_N = 2000
BLK_E = 2560


def _deg_finalize_body(degp, out):
  d = jnp.sum(degp[...], axis=(0, 1))
  out[...] = 1.0 / jnp.maximum(d, 1.0)


def _deg_finalize(degp):
  return pl.pallas_call(
      _deg_finalize_body,
      out_shape=jax.ShapeDtypeStruct((DROW, D), jnp.float32),
  )(degp)


def _sage_tc_body(relu, aggp, recip, h, wself, wneigh, b, out):
  mean = (aggp[0] + aggp[1]) * recip[...]
  r = (jnp.dot(h[...], wself[...], preferred_element_type=jnp.float32)
       + jnp.dot(mean, wneigh[...], preferred_element_type=jnp.float32)
       + b[...])
  out[...] = jnp.maximum(r, 0.0) if relu else r


def _sage_tc(aggp, recip, h, wself, wneigh, b, relu):
  grid = (N // BLK_N,)
  return pl.pallas_call(
      functools.partial(_sage_tc_body, relu),
      grid=grid,
      in_specs=[
          pl.BlockSpec((NC, BLK_N, D), lambda m: (0, m, 0)),
          pl.BlockSpec((BLK_N, 1), lambda m: (m, 0)),
          pl.BlockSpec((BLK_N, D), lambda m: (m, 0)),
          pl.BlockSpec((D, D), lambda m: (0, 0)),
          pl.BlockSpec((D, D), lambda m: (0, 0)),
          pl.BlockSpec((1, D), lambda m: (0, 0)),
      ],
      out_specs=pl.BlockSpec((BLK_N, D), lambda m: (m, 0)),
      out_shape=jax.ShapeDtypeStruct((N, D), jnp.float32),
  )(aggp, recip, h, wself, wneigh, b)


def _sage_final_body(aggp, recip, h, wself, wneigh, b, w1t, w1b, a_out, b_out):
  mean = (aggp[0] + aggp[1]) * recip[...]
  h3 = (jnp.dot(h[...], wself[...], preferred_element_type=jnp.float32)
        + jnp.dot(mean, wneigh[...], preferred_element_type=jnp.float32)
        + b[...])
  a_out[...] = jnp.dot(h3, w1t[...], preferred_element_type=jnp.float32)
  b_out[...] = jnp.dot(h3, w1b[...], preferred_element_type=jnp.float32)


def _sage_final_tc(aggp, recip, h, wself, wneigh, b, w1t, w1b):
  grid = (N // BLK_N,)
  return pl.pallas_call(
      _sage_final_body,
      grid=grid,
      in_specs=[
          pl.BlockSpec((NC, BLK_N, D), lambda m: (0, m, 0)),
          pl.BlockSpec((BLK_N, 1), lambda m: (m, 0)),
          pl.BlockSpec((BLK_N, D), lambda m: (m, 0)),
          pl.BlockSpec((D, D), lambda m: (0, 0)),
          pl.BlockSpec((D, D), lambda m: (0, 0)),
          pl.BlockSpec((1, D), lambda m: (0, 0)),
          pl.BlockSpec((D, D), lambda m: (0, 0)),
          pl.BlockSpec((D, D), lambda m: (0, 0)),
      ],
      out_specs=[
          pl.BlockSpec((BLK_N, D), lambda m: (m, 0)),
          pl.BlockSpec((BLK_N, D), lambda m: (m, 0)),
      ],
      out_shape=[
          jax.ShapeDtypeStruct((N, D), jnp.float32),
          jax.ShapeDtypeStruct((N, D), jnp.float32),
      ],
  )(aggp, recip, h, wself, wneigh, b, w1t, w1b)


def _mlp_body(s, b1, w2, b2, w3t, b3, out):
  z1 = jnp.maximum(s[...] + b1[...], 0.0)
  z2 = jnp.maximum(
      jnp.dot(z1, w2[...], preferred_element_type=jnp.float32) + b2[...], 0.0)
  out[...] = lax.dot_general(
      w3t[...], z2, (((1,), (1,)), ((), ())),
      preferred_element_type=jnp.float32) + b3[...]


def _mlp_tc(s, b1, w2, b2, w3t, b3):
  grid = (E // BLK_E,)
  return pl.pallas_call(
      _mlp_body,
      grid=grid,
      in_specs=[
          pl.BlockSpec((BLK_E, D), lambda m: (m, 0)),
          pl.BlockSpec((1, D), lambda m: (0, 0)),
          pl.BlockSpec((D, D), lambda m: (0, 0)),
          pl.BlockSpec((1, D), lambda m: (0, 0)),
          pl.BlockSpec((2, D), lambda m: (0, 0)),
          pl.BlockSpec((2, 1), lambda m: (0, 0)),
      ],
      out_specs=pl.BlockSpec((2, BLK_E), lambda m: (0, m)),
      out_shape=jax.ShapeDtypeStruct((2, E), jnp.float32),
  )(s, b1, w2, b2, w3t, b3)


def kernel(x, edge_index, pos_edge_index, neg_edge_index,
           W_self_0, W_neigh_0, b_0, W_self_1, W_neigh_1, b_1,
           W_self_2, W_neigh_2, b_2,
           W_mlp1, b_mlp1, W_mlp2, b_mlp2, W_mlp3, b_mlp3):
  b0 = b_0.reshape(1, D)
  b1l = b_1.reshape(1, D)
  b2l = b_2.reshape(1, D)
  bm1 = b_mlp1.reshape(1, D)
  bm2 = b_mlp2.reshape(1, D)
  bm3 = b_mlp3.reshape(2, 1)
  w3t = W_mlp3.T
  w1t = W_mlp1[:D]
  w1b = W_mlp1[D:]

  src = edge_index[0]
  dst = edge_index[1]
  pu, pv = pos_edge_index[0], pos_edge_index[1]
  nu, nv = neg_edge_index[0], neg_edge_index[1]

  aggp, degp = _sc_agg_deg(src, dst, x)
  recip = _deg_finalize(degp).reshape(N_PAD, 1)
  h = _sage_tc(aggp, recip, x, W_self_0, W_neigh_0, b0, True)
  aggp = _sc_agg(src, dst, h)[0]
  h = _sage_tc(aggp, recip, h, W_self_1, W_neigh_1, b1l, True)
  aggp = _sc_agg(src, dst, h)[0]
  A, B = _sage_final_tc(aggp, recip, h, W_self_2, W_neigh_2, b2l, w1t, w1b)

  s_pos = _sc_pair_gather(pu, pv, A, B)
  s_neg = _sc_pair_gather(nu, nv, A, B)
  pos = _mlp_tc(s_pos, bm1, W_mlp2, bm2, w3t, bm3)
  neg = _mlp_tc(s_neg, bm1, W_mlp2, bm2, w3t, bm3)
  return (pos.T, neg.T)
